# baseline (speedup 1.0000x reference)
import functools

import jax
import jax.numpy as jnp
from jax import lax
from jax.experimental import pallas as pl
from jax.experimental.pallas import tpu as pltpu
from jax.experimental.pallas import tpu_sc as plsc

NC, NS, L = 2, 16, 16
NW = NC * NS
B, C, T = 1024, 1000, 10
CP = 1008
NCHUNK = CP // L
ROWS_PER_W = B // NW
NEG = -1e30

_mesh = plsc.VectorSubcoreMesh(
    core_axis_name="c", subcore_axis_name="s", num_cores=NC, num_subcores=NS
)


@functools.partial(
    pl.kernel,
    out_type=(
        jax.ShapeDtypeStruct((NW, L), jnp.float32),
        jax.ShapeDtypeStruct((NW, L), jnp.float32),
    ),
    mesh=_mesh,
    compiler_params=pltpu.CompilerParams(needs_layout_passes=False),
    scratch_types=[
        pltpu.VMEM((CP,), jnp.float32),
        pltpu.VMEM((CP,), jnp.float32),
        pltpu.VMEM((ROWS_PER_W * T,), jnp.int32),
        pltpu.VMEM((L,), jnp.float32),
        pltpu.VMEM((L,), jnp.float32),
    ],
)
def _loss_kernel(x_hbm, m_hbm, tgt_hbm, out_a, out_t, xbuf, mbuf, tbuf, avec, tvec):
    wid = lax.axis_index("s") * NC + lax.axis_index("c")
    base_row = wid * ROWS_PER_W

    pltpu.sync_copy(tgt_hbm.at[pl.ds(base_row * T, ROWS_PER_W * T)], tbuf)

    lane = lax.iota(jnp.int32, L)
    tail_keep = lane < (C - (NCHUNK - 1) * L)

    def row_body(r, carry):
        accs, thsum = carry
        accs = list(accs)
        row = base_row + r
        pltpu.sync_copy(x_hbm.at[pl.ds(row * C, C)], xbuf.at[pl.ds(0, C)])
        pltpu.sync_copy(m_hbm.at[pl.ds(row * C, C)], mbuf.at[pl.ds(0, C)])

        thetas = []
        for t in range(T):
            tidx = plsc.load_gather(tbuf, [jnp.full((L,), r * T + t, jnp.int32)])
            xt = plsc.load_gather(xbuf, [tidx])
            mt = plsc.load_gather(mbuf, [tidx])
            th = xt - mt - 1.0
            thetas.append(th)
            thsum = thsum + th

        for k in range(NCHUNK):
            s = xbuf[pl.ds(k * L, L)] - mbuf[pl.ds(k * L, L)]
            if k == NCHUNK - 1:
                s = jnp.where(tail_keep, s, NEG)
            for t in range(T):
                accs[t] = accs[t] + jnp.maximum(s, thetas[t])
        return tuple(accs), thsum

    zero = jnp.zeros((L,), jnp.float32)
    accs, thsum = lax.fori_loop(
        0, ROWS_PER_W, row_body, (tuple(zero for _ in range(T)), zero)
    )
    acc = accs[0]
    for t in range(1, T):
        acc = acc + accs[t]

    avec[...] = acc
    tvec[...] = thsum
    pltpu.sync_copy(avec, out_a.at[wid])
    pltpu.sync_copy(tvec, out_t.at[wid])


def kernel(input_data, target, adaptive_margin):
    x = input_data.reshape(-1)
    m = adaptive_margin.reshape(-1)
    tgt = target.reshape(-1).astype(jnp.int32)
    out_a, out_t = _loss_kernel(x, m, tgt)
    total = jnp.sum(out_a) - CP * jnp.sum(out_t[:, 0]) - jnp.float32(B * T)
    return total / jnp.float32(C)


# speedup vs baseline: 1.4866x; 1.4866x over previous
import functools

import jax
import jax.numpy as jnp
from jax import lax
from jax.experimental import pallas as pl
from jax.experimental.pallas import tpu as pltpu
from jax.experimental.pallas import tpu_sc as plsc

NC, NS, L = 2, 16, 16
NW = NC * NS
B, C, T = 1024, 1000, 10
CP = 1008
NCHUNK = CP // L
ROWS_PER_W = B // NW
NEG = -1e30

_mesh = plsc.VectorSubcoreMesh(
    core_axis_name="c", subcore_axis_name="s", num_cores=NC, num_subcores=NS
)


@functools.partial(
    pl.kernel,
    out_type=(
        jax.ShapeDtypeStruct((NW, L), jnp.float32),
        jax.ShapeDtypeStruct((NW, L), jnp.float32),
    ),
    mesh=_mesh,
    compiler_params=pltpu.CompilerParams(needs_layout_passes=False),
    scratch_types=[
        pltpu.VMEM((2 * CP,), jnp.float32),
        pltpu.VMEM((2 * CP,), jnp.float32),
        pltpu.VMEM((ROWS_PER_W * T,), jnp.int32),
        pltpu.VMEM((L,), jnp.float32),
        pltpu.VMEM((L,), jnp.float32),
        pltpu.SemaphoreType.DMA,
        pltpu.SemaphoreType.DMA,
    ],
)
def _loss_kernel(x_hbm, m_hbm, tgt_hbm, out_a, out_t, xbuf, mbuf, tbuf, avec, tvec,
                 sem0, sem1):
    wid = lax.axis_index("s") * NC + lax.axis_index("c")
    base_row = wid * ROWS_PER_W
    last_row = base_row + ROWS_PER_W - 1

    pltpu.sync_copy(tgt_hbm.at[pl.ds(base_row * T, ROWS_PER_W * T)], tbuf)

    lane = lax.iota(jnp.int32, L)
    tail_keep = lane < (C - (NCHUNK - 1) * L)

    def dma_row(row, off, sem):
        pltpu.async_copy(x_hbm.at[pl.ds(row * C, C)], xbuf.at[pl.ds(off, C)], sem)
        pltpu.async_copy(m_hbm.at[pl.ds(row * C, C)], mbuf.at[pl.ds(off, C)], sem)

    def wait_set(off, sem):
        pltpu.make_async_copy(
            x_hbm.at[pl.ds(0, C)], xbuf.at[pl.ds(off, C)], sem).wait()
        pltpu.make_async_copy(
            m_hbm.at[pl.ds(0, C)], mbuf.at[pl.ds(off, C)], sem).wait()

    def row_compute(r, off, accs, thsum):
        thetas = []
        for t in range(T):
            tidx = plsc.load_gather(tbuf, [jnp.full((L,), r * T + t, jnp.int32)])
            xt = plsc.load_gather(xbuf, [tidx + off])
            mt = plsc.load_gather(mbuf, [tidx + off])
            th = xt - mt - 1.0
            thetas.append(th)
            thsum = thsum + th

        for k in range(NCHUNK):
            s = xbuf[pl.ds(off + k * L, L)] - mbuf[pl.ds(off + k * L, L)]
            if k == NCHUNK - 1:
                s = jnp.where(tail_keep, s, NEG)
            for t in range(T):
                accs[t] = accs[t] + jnp.maximum(s, thetas[t])
        return accs, thsum

    dma_row(base_row, 0, sem0)
    dma_row(base_row + 1, CP, sem1)

    def pair_body(g, carry):
        accs, thsum = carry
        accs = list(accs)
        r0 = 2 * g

        wait_set(0, sem0)
        accs, thsum = row_compute(r0, 0, accs, thsum)
        dma_row(jnp.minimum(base_row + r0 + 2, last_row), 0, sem0)

        wait_set(CP, sem1)
        accs, thsum = row_compute(r0 + 1, CP, accs, thsum)
        dma_row(jnp.minimum(base_row + r0 + 3, last_row), CP, sem1)

        return tuple(accs), thsum

    zero = jnp.zeros((L,), jnp.float32)
    accs, thsum = lax.fori_loop(
        0, ROWS_PER_W // 2, pair_body, (tuple(zero for _ in range(T)), zero)
    )

    wait_set(0, sem0)
    wait_set(CP, sem1)

    acc = accs[0]
    for t in range(1, T):
        acc = acc + accs[t]

    avec[...] = acc
    tvec[...] = thsum
    pltpu.sync_copy(avec, out_a.at[wid])
    pltpu.sync_copy(tvec, out_t.at[wid])


def kernel(input_data, target, adaptive_margin):
    x = input_data.reshape(-1)
    m = adaptive_margin.reshape(-1)
    tgt = target.reshape(-1).astype(jnp.int32)
    out_a, out_t = _loss_kernel(x, m, tgt)
    total = jnp.sum(out_a) - CP * jnp.sum(out_t[:, 0]) - jnp.float32(B * T)
    return total / jnp.float32(C)
